# Initial kernel scaffold; baseline (speedup 1.0000x reference)
import jax
import jax.numpy as jnp
from jax.experimental import pallas as pl


def kernel(x, emb):
    raise NotImplementedError("write your pallas kernel here")


# speedup vs baseline: 1.4404x; 1.4404x over previous
import functools

import numpy as np
import jax
import jax.numpy as jnp
from jax import lax
from jax.experimental import pallas as pl
from jax.experimental.pallas import tpu as pltpu
from jax.experimental.pallas import tpu_sc as plsc

VOCAB = 100000
D = 768
SEQ = 4096
BATCH = 4
BFLAT = BATCH * SEQ

NC, NS = 2, 16
NW = NC * NS
BPW = BFLAT // NW
K = 32
NCHUNK = BPW // K
LANES = 16


def _pos_encoding() -> np.ndarray:
    pos = np.arange(SEQ, dtype=np.float64)[:, None]
    i2 = np.arange(0, D, 2, dtype=np.float64)
    enc = np.zeros((SEQ, D), dtype=np.float32)
    enc[:, 0::2] = np.sin(pos / 10000 ** (i2 / D)).astype(np.float32)
    enc[:, 1::2] = np.cos(pos / 10000 ** (i2 / D)).astype(np.float32)
    return enc


_POS = _pos_encoding()


def _body(x_hbm, pos_hbm, emb_hbm, out_hbm, idx_v, rows_v, pos_v, gsem):
    wid = lax.axis_index("s") * NC + lax.axis_index("c")
    base = wid * BPW
    pbase = lax.rem(base, SEQ)

    pltpu.sync_copy(x_hbm.at[pl.ds(base, BPW)], idx_v)

    def chunk(c, _):
        cp = pltpu.async_copy(emb_hbm.at[idx_v.at[pl.ds(c * K, K)]], rows_v, gsem)
        pltpu.sync_copy(pos_hbm.at[pl.ds(pbase + c * K, K)], pos_v)
        cp.wait()

        def row(r, _):
            for j in range(D // LANES):
                v = pos_v[r, pl.ds(j * LANES, LANES)]
                plsc.addupdate(rows_v.at[r, pl.ds(j * LANES, LANES)], v)
            return 0

        lax.fori_loop(0, K, row, 0, unroll=False)
        pltpu.sync_copy(rows_v, out_hbm.at[pl.ds(base + c * K, K)])
        return 0

    lax.fori_loop(0, NCHUNK, chunk, 0, unroll=False)


@jax.jit
def _run(xf, emb):
    mesh = plsc.VectorSubcoreMesh(core_axis_name="c", subcore_axis_name="s",
                                  num_cores=NC, num_subcores=NS)
    pos = jnp.asarray(_POS)
    return pl.kernel(
        _body,
        out_type=jax.ShapeDtypeStruct((BFLAT, D), jnp.float32),
        mesh=mesh,
        scratch_types=[
            pltpu.VMEM((BPW,), jnp.int32),
            pltpu.VMEM((K, D), jnp.float32),
            pltpu.VMEM((K, D), jnp.float32),
            pltpu.SemaphoreType.DMA,
        ],
    )(xf, pos, emb)


def kernel(x, emb):
    xf = x.reshape(-1).astype(jnp.int32)
    out = _run(xf, emb)
    return out.reshape(BATCH, SEQ, D)
